# baseline (speedup 1.0000x reference)
import functools

import jax
import jax.numpy as jnp
from jax import lax
from jax.experimental import pallas as pl
from jax.experimental.pallas import tpu as pltpu
from jax.experimental.pallas import tpu_sc as plsc

_N_NODES = 10000
_N_EDGES = 320000
_N_FACES = 160000
_D = 128
_ALPHA = 0.1

_NC = 2
_NS = 16

_BLK = 2000


_A_CH = 200
_EPC = _N_EDGES // _NC
_EPT = _EPC // _NS
_ROWS_PT = _N_NODES // _NS


def _node_scatter_body(x1_hbm, src_hbm, dst_hbm, yp_hbm, y_acc, xbuf, dstbuf, srcbuf):
    c = lax.axis_index("c")
    s = lax.axis_index("s")
    base = c * _EPC + s * _EPT

    def zrow(r, carry2):
        for c8 in range(_D // 16):
            xbuf[r, pl.ds(c8 * 16, 16)] = jnp.zeros((16,), jnp.float32)
        return carry2

    lax.fori_loop(0, 104, zrow, 0)
    for j in range(6):
        pltpu.sync_copy(xbuf.at[pl.ds(0, 104)],
                        y_acc.at[pl.ds(s * 624 + j * 104, 104)])

    @pl.when(s == _NS - 1)
    def _ztail():
        pltpu.sync_copy(xbuf.at[pl.ds(0, _N_NODES - 9984)],
                        y_acc.at[pl.ds(9984, _N_NODES - 9984)])

    plsc.subcore_barrier()

    def chunk(i, carry):
        off = base + i * _A_CH
        pltpu.sync_copy(x1_hbm.at[pl.ds(off, _A_CH)], xbuf)
        pltpu.sync_copy(dst_hbm.at[pl.ds(off, _A_CH)], dstbuf)
        pltpu.sync_copy(src_hbm.at[pl.ds(off, _A_CH)], srcbuf)
        pltpu.sync_copy(xbuf, y_acc.at[dstbuf], add=True)

        def neg_row(r, carry2):
            for c8 in range(_D // 16):
                xbuf[r, pl.ds(c8 * 16, 16)] = -xbuf[r, pl.ds(c8 * 16, 16)]
            return carry2

        lax.fori_loop(0, _A_CH, neg_row, 0)
        pltpu.sync_copy(xbuf, y_acc.at[srcbuf], add=True)
        return carry

    lax.fori_loop(0, _EPT // _A_CH, chunk, 0)
    plsc.subcore_barrier()
    pltpu.sync_copy(y_acc.at[pl.ds(s * 624, 624)],
                    yp_hbm.at[c, pl.ds(s * 624, 624)])

    @pl.when(s == _NS - 1)
    def _tail():
        pltpu.sync_copy(y_acc.at[pl.ds(9984, _N_NODES - 9984)],
                        yp_hbm.at[c, pl.ds(9984, _N_NODES - 9984)])


@jax.jit
def _node_scatter(x1, src, dst):
    mesh = plsc.VectorSubcoreMesh(core_axis_name="c", subcore_axis_name="s",
                                  num_cores=_NC, num_subcores=_NS)
    return pl.kernel(
        _node_scatter_body,
        out_type=jax.ShapeDtypeStruct((_NC, _N_NODES, _D), jnp.float32),
        mesh=mesh,
        scratch_types=[
            pltpu.VMEM_SHARED((_N_NODES, _D), jnp.float32),
            pltpu.VMEM((_A_CH, _D), jnp.float32),
            pltpu.VMEM((_A_CH,), jnp.int32),
            pltpu.VMEM((_A_CH,), jnp.int32),
        ],
    )(x1, src, dst)


_C_CH = 200
_FPT = _N_FACES // (_NC * _NS)


def _face_gather_body(x1_hbm, f0_hbm, f1_hbm, f2_hbm, u2_hbm, rank_hbm, w_hbm,
                      f0b, f1b, f2b, r0b, r1b, r2b, A, T, B,
                      sem, sem2, sem3, sem4):
    c = lax.axis_index("c")
    s = lax.axis_index("s")
    wid = c * _NS + s
    base = wid * _FPT
    pltpu.sync_copy(f0_hbm.at[pl.ds(base, _FPT)], f0b)
    pltpu.sync_copy(f1_hbm.at[pl.ds(base, _FPT)], f1b)
    pltpu.sync_copy(f2_hbm.at[pl.ds(base, _FPT)], f2b)
    pltpu.sync_copy(rank_hbm.at[pl.ds(base, _FPT)], r0b)
    pltpu.sync_copy(rank_hbm.at[pl.ds(_N_FACES + base, _FPT)], r1b)
    pltpu.sync_copy(rank_hbm.at[pl.ds(2 * _N_FACES + base, _FPT)], r2b)

    def chunk(i, carry):
        off = i * _C_CH
        cp0 = pltpu.async_copy(x1_hbm.at[f0b.at[pl.ds(off, _C_CH)]], A, sem)
        cp1 = pltpu.async_copy(x1_hbm.at[f1b.at[pl.ds(off, _C_CH)]], T, sem2)
        pltpu.sync_copy(u2_hbm.at[pl.ds(base + off, _C_CH)], B)
        cp0.wait()
        pltpu.async_copy(x1_hbm.at[f2b.at[pl.ds(off, _C_CH)]], A, sem,
                         add=True).wait()
        cp1.wait()

        def rowop(r, carry2):
            for c8 in range(_D // 16):
                sl = pl.ds(c8 * 16, 16)
                B[r, sl] = B[r, sl] - _ALPHA * (A[r, sl] - T[r, sl])
            return carry2

        lax.fori_loop(0, _C_CH, rowop, 0)
        cw0 = pltpu.async_copy(B, w_hbm.at[r0b.at[pl.ds(off, _C_CH)]], sem3)
        cw2 = pltpu.async_copy(B, w_hbm.at[r2b.at[pl.ds(off, _C_CH)]], sem4)
        cw0.wait()
        cw2.wait()

        def rowneg(r, carry2):
            for c8 in range(_D // 16):
                sl = pl.ds(c8 * 16, 16)
                B[r, sl] = -B[r, sl]
            return carry2

        lax.fori_loop(0, _C_CH, rowneg, 0)
        pltpu.async_copy(B, w_hbm.at[r1b.at[pl.ds(off, _C_CH)]], sem3).wait()
        return carry

    lax.fori_loop(0, _FPT // _C_CH, chunk, 0)


@jax.jit
def _face_gather(x1, f0, f1, f2, u2, rank):
    mesh = plsc.VectorSubcoreMesh(core_axis_name="c", subcore_axis_name="s",
                                  num_cores=_NC, num_subcores=_NS)
    return pl.kernel(
        _face_gather_body,
        out_type=jax.ShapeDtypeStruct((3 * _N_FACES, _D), jnp.float32),
        mesh=mesh,
        scratch_types=[
            pltpu.VMEM((_FPT,), jnp.int32),
            pltpu.VMEM((_FPT,), jnp.int32),
            pltpu.VMEM((_FPT,), jnp.int32),
            pltpu.VMEM((_FPT,), jnp.int32),
            pltpu.VMEM((_FPT,), jnp.int32),
            pltpu.VMEM((_FPT,), jnp.int32),
            pltpu.VMEM((_C_CH, _D), jnp.float32),
            pltpu.VMEM((_C_CH, _D), jnp.float32),
            pltpu.VMEM((_C_CH, _D), jnp.float32),
            pltpu.SemaphoreType.DMA,
            pltpu.SemaphoreType.DMA,
            pltpu.SemaphoreType.DMA,
            pltpu.SemaphoreType.DMA,
        ],
    )(x1, f0, f1, f2, u2, rank)


_G_CH = 200
_G_EPT = _N_EDGES // (_NC * _NS)


def _edge_gather_body(v_hbm, src_hbm, dst_hbm, g_hbm,
                      db, sb, VD, VS, semd, sems):
    c = lax.axis_index("c")
    s = lax.axis_index("s")
    wid = c * _NS + s
    base = wid * _G_EPT

    def chunk(i, carry):
        off = base + i * _G_CH
        pltpu.sync_copy(dst_hbm.at[pl.ds(off, _G_CH)], db)
        pltpu.sync_copy(src_hbm.at[pl.ds(off, _G_CH)], sb)
        cpd = pltpu.async_copy(v_hbm.at[db], VD, semd)
        cps = pltpu.async_copy(v_hbm.at[sb], VS, sems)
        cpd.wait()
        cps.wait()

        def rowop(r, carry2):
            for c8 in range(_D // 16):
                sl = pl.ds(c8 * 16, 16)
                VD[r, sl] = VD[r, sl] - VS[r, sl]
            return carry2

        lax.fori_loop(0, _G_CH, rowop, 0)
        pltpu.sync_copy(VD, g_hbm.at[pl.ds(off, _G_CH)])
        return carry

    lax.fori_loop(0, _G_EPT // _G_CH, chunk, 0)


@jax.jit
def _edge_gather(v, src, dst):
    mesh = plsc.VectorSubcoreMesh(core_axis_name="c", subcore_axis_name="s",
                                  num_cores=_NC, num_subcores=_NS)
    return pl.kernel(
        _edge_gather_body,
        out_type=jax.ShapeDtypeStruct((_N_EDGES, _D), jnp.float32),
        mesh=mesh,
        scratch_types=[
            pltpu.VMEM((_G_CH,), jnp.int32),
            pltpu.VMEM((_G_CH,), jnp.int32),
            pltpu.VMEM((_G_CH, _D), jnp.float32),
            pltpu.VMEM((_G_CH, _D), jnp.float32),
            pltpu.SemaphoreType.DMA,
            pltpu.SemaphoreType.DMA,
        ],
    )(v, src, dst)


_DC = 12800
_NCHUNK = _N_EDGES // _DC
_FPAD = 160256
_FPT_D = _FPAD // (_NC * _NS)
_SCAN_BLKS = [32] * 9 + [25]
_DSIGNS = (1.0, -1.0, 1.0)


def _edge_scatter_body(f0_hbm, f1_hbm, f2_hbm, w_hbm, e_hbm,
                       e_acc, fb0, fb1, fb2, locl, fidl, rowbuf, zbuf, cntb, sem):
    c = lax.axis_index("c")
    s = lax.axis_index("s")
    wid = c * _NS + s
    fbase = wid * _FPT_D
    pltpu.sync_copy(f0_hbm.at[pl.ds(fbase, _FPT_D)], fb0)
    pltpu.sync_copy(f1_hbm.at[pl.ds(fbase, _FPT_D)], fb1)
    pltpu.sync_copy(f2_hbm.at[pl.ds(fbase, _FPT_D)], fb2)

    def zrow(r, carry2):
        for c8 in range(_D // 16):
            zbuf[r, pl.ds(c8 * 16, 16)] = jnp.zeros((16,), jnp.float32)
        return carry2

    lax.fori_loop(0, 32, zrow, 0)
    iota16 = lax.iota(jnp.int32, 16)
    rows_pt = _DC // _NS

    def flush(k, cnt):
        plsc.store_scatter(locl, [cnt + iota16],
                           jnp.full((16,), _DC, jnp.int32) + iota16)
        plsc.store_scatter(fidl, [cnt + iota16], iota16)
        nb = lax.shift_right_logical(cnt + 15, 4)

        def batch(jb, carry3):
            fidv = fidl[pl.ds(jb * 16, 16)]
            locv = locl[pl.ds(jb * 16, 16)]
            pltpu.async_copy(w_hbm.at[fidv], rowbuf, sem).wait()
            if k == 1:
                for r in range(16):
                    for c8 in range(_D // 16):
                        sl = pl.ds(c8 * 16, 16)
                        rowbuf[r, sl] = -rowbuf[r, sl]
            pltpu.sync_copy(rowbuf, e_acc.at[locv], add=True)
            return carry3

        lax.fori_loop(0, nb, batch, 0)

    n_pass = jnp.where(c == 0, 13, _NCHUNK - 13)

    def pass_body(i, carry):
        ch = i + c * 13
        base = ch * _DC
        for j in range(rows_pt // 32):
            pltpu.sync_copy(zbuf, e_acc.at[pl.ds(s * rows_pt + j * 32, 32)])
        plsc.subcore_barrier()
        for k, fb in ((0, fb0), (1, fb1), (2, fb2)):
            voff = 0
            for nv in _SCAN_BLKS:
                def scan_iter(j, cnt, _voff=voff, _fb=fb):
                    idx = _fb[pl.ds((_voff + j) * 16, 16)]
                    loc = idx - base
                    mask = (idx >= base) & (loc < _DC)
                    cs = plsc.cumsum(mask.astype(jnp.int32))
                    pos = jnp.where(mask, cnt + cs - 1, 543)
                    plsc.store_scatter(locl, [pos], loc)
                    fidv = fbase + (_voff + j) * 16 + iota16
                    plsc.store_scatter(fidl, [pos], fidv)
                    return cnt + cs[15]

                cnt = lax.fori_loop(0, nv, scan_iter, jnp.int32(0))
                flush(k, cnt)
                voff += nv
        plsc.subcore_barrier()
        pltpu.sync_copy(e_acc.at[pl.ds(s * rows_pt, rows_pt)],
                        e_hbm.at[pl.ds(base + s * rows_pt, rows_pt)])
        return carry

    lax.fori_loop(0, n_pass, pass_body, 0)


@jax.jit
def _edge_scatter(f0p, f1p, f2p, w):
    mesh = plsc.VectorSubcoreMesh(core_axis_name="c", subcore_axis_name="s",
                                  num_cores=_NC, num_subcores=_NS)
    return pl.kernel(
        _edge_scatter_body,
        out_type=jax.ShapeDtypeStruct((_N_EDGES, _D), jnp.float32),
        mesh=mesh,
        scratch_types=[
            pltpu.VMEM_SHARED((_DC + 16, _D), jnp.float32),
            pltpu.VMEM((_FPT_D,), jnp.int32),
            pltpu.VMEM((_FPT_D,), jnp.int32),
            pltpu.VMEM((_FPT_D,), jnp.int32),
            pltpu.VMEM((544,), jnp.int32),
            pltpu.VMEM((544,), jnp.int32),
            pltpu.VMEM((16, _D), jnp.float32),
            pltpu.VMEM((32, _D), jnp.float32),
            pltpu.VMEM((16,), jnp.int32),
            pltpu.SemaphoreType.DMA,
        ],
    )(f0p, f1p, f2p, w)


def _v_body(x0_ref, yp_ref, w_ref, o_ref):
    u0 = jnp.dot(x0_ref[...], w_ref[...], preferred_element_type=jnp.float32)
    o_ref[...] = u0 - _ALPHA * (yp_ref[0] + yp_ref[1])


@jax.jit
def _v_table(x0, yp, W_grad_t):
    n = x0.shape[0]
    return pl.pallas_call(
        _v_body,
        grid=(n // _BLK,),
        in_specs=[
            pl.BlockSpec((_BLK, _D), lambda i: (i, 0)),
            pl.BlockSpec((_NC, _BLK, _D), lambda i: (0, i, 0)),
            pl.BlockSpec((_D, _D), lambda i: (0, 0)),
        ],
        out_specs=pl.BlockSpec((_BLK, _D), lambda i: (i, 0)),
        out_shape=jax.ShapeDtypeStruct((n, _D), jnp.float32),
    )(x0, yp, W_grad_t)


def _mm_body(x_ref, w_ref, o_ref):
    o_ref[...] = jnp.dot(x_ref[...], w_ref[...], preferred_element_type=jnp.float32)


@jax.jit
def _matmul(x, Wt):
    n = x.shape[0]
    return pl.pallas_call(
        _mm_body,
        grid=(n // _BLK,),
        in_specs=[
            pl.BlockSpec((_BLK, _D), lambda i: (i, 0)),
            pl.BlockSpec((_D, _D), lambda i: (0, 0)),
        ],
        out_specs=pl.BlockSpec((_BLK, _D), lambda i: (i, 0)),
        out_shape=jax.ShapeDtypeStruct((n, _D), jnp.float32),
    )(x, Wt)


def _final_body(x1_ref, e_ref, w_ref, b_ref, o_ref):
    acc = jnp.dot(x1_ref[...], w_ref[...], preferred_element_type=jnp.float32)
    o_ref[...] = jnp.maximum(acc + e_ref[...] + b_ref[...], 0.0)


@jax.jit
def _final_matmul(x1, e, W_self_t, bias):
    n = x1.shape[0]
    return pl.pallas_call(
        _final_body,
        grid=(n // _BLK,),
        in_specs=[
            pl.BlockSpec((_BLK, _D), lambda i: (i, 0)),
            pl.BlockSpec((_BLK, _D), lambda i: (i, 0)),
            pl.BlockSpec((_D, _D), lambda i: (0, 0)),
            pl.BlockSpec((1, _D), lambda i: (0, 0)),
        ],
        out_specs=pl.BlockSpec((_BLK, _D), lambda i: (i, 0)),
        out_shape=jax.ShapeDtypeStruct((n, _D), jnp.float32),
    )(x1, e, W_self_t, bias)


def kernel(x1, x0, x2, edge_index, face_edge_index, W_self, W_grad, W_curl, bias):
    src = edge_index[0]
    dst = edge_index[1]
    fe0, fe1, fe2 = face_edge_index[0], face_edge_index[1], face_edge_index[2]

    yp = _node_scatter(x1, src, dst)
    v = _v_table(x0, yp, W_grad.T)
    g = _edge_gather(v, src, dst)

    all_idx = jnp.concatenate([fe0, fe1, fe2])
    n_all = all_idx.shape[0]
    iota = lax.iota(jnp.int32, n_all)
    sorted_idx, perm = lax.sort_key_val(all_idx, iota)
    rank = lax.sort_key_val(perm, iota)[1]

    u2 = _matmul(x2, W_curl.T)
    ws_sorted = _face_gather(x1, fe0, fe1, fe2, u2, rank)

    e = g.at[sorted_idx].add(ws_sorted, indices_are_sorted=True)

    return _final_matmul(x1, e, W_self.T, bias.reshape(1, _D))


# speedup vs baseline: 1.3219x; 1.3219x over previous
import functools

import jax
import jax.numpy as jnp
from jax import lax
from jax.experimental import pallas as pl
from jax.experimental.pallas import tpu as pltpu
from jax.experimental.pallas import tpu_sc as plsc

_N_NODES = 10000
_N_EDGES = 320000
_N_FACES = 160000
_D = 128
_ALPHA = 0.1

_NC = 2
_NS = 16

_BLK = 2000


_A_CH = 200
_EPC = _N_EDGES // _NC
_EPT = _EPC // _NS
_ROWS_PT = _N_NODES // _NS


def _node_scatter_body(x1_hbm, src_hbm, dst_hbm, yp_hbm, y_acc, xbuf, dstbuf, srcbuf):
    c = lax.axis_index("c")
    s = lax.axis_index("s")
    base = c * _EPC + s * _EPT

    def zrow(r, carry2):
        for c8 in range(_D // 16):
            xbuf[r, pl.ds(c8 * 16, 16)] = jnp.zeros((16,), jnp.float32)
        return carry2

    lax.fori_loop(0, 104, zrow, 0)
    for j in range(6):
        pltpu.sync_copy(xbuf.at[pl.ds(0, 104)],
                        y_acc.at[pl.ds(s * 624 + j * 104, 104)])

    @pl.when(s == _NS - 1)
    def _ztail():
        pltpu.sync_copy(xbuf.at[pl.ds(0, _N_NODES - 9984)],
                        y_acc.at[pl.ds(9984, _N_NODES - 9984)])

    plsc.subcore_barrier()

    def chunk(i, carry):
        off = base + i * _A_CH
        pltpu.sync_copy(x1_hbm.at[pl.ds(off, _A_CH)], xbuf)
        pltpu.sync_copy(dst_hbm.at[pl.ds(off, _A_CH)], dstbuf)
        pltpu.sync_copy(src_hbm.at[pl.ds(off, _A_CH)], srcbuf)
        pltpu.sync_copy(xbuf, y_acc.at[dstbuf], add=True)

        def neg_row(r, carry2):
            for c8 in range(_D // 16):
                xbuf[r, pl.ds(c8 * 16, 16)] = -xbuf[r, pl.ds(c8 * 16, 16)]
            return carry2

        lax.fori_loop(0, _A_CH, neg_row, 0)
        pltpu.sync_copy(xbuf, y_acc.at[srcbuf], add=True)
        return carry

    lax.fori_loop(0, _EPT // _A_CH, chunk, 0)
    plsc.subcore_barrier()
    pltpu.sync_copy(y_acc.at[pl.ds(s * 624, 624)],
                    yp_hbm.at[c, pl.ds(s * 624, 624)])

    @pl.when(s == _NS - 1)
    def _tail():
        pltpu.sync_copy(y_acc.at[pl.ds(9984, _N_NODES - 9984)],
                        yp_hbm.at[c, pl.ds(9984, _N_NODES - 9984)])


@jax.jit
def _node_scatter(x1, src, dst):
    mesh = plsc.VectorSubcoreMesh(core_axis_name="c", subcore_axis_name="s",
                                  num_cores=_NC, num_subcores=_NS)
    return pl.kernel(
        _node_scatter_body,
        out_type=jax.ShapeDtypeStruct((_NC, _N_NODES, _D), jnp.float32),
        mesh=mesh,
        scratch_types=[
            pltpu.VMEM_SHARED((_N_NODES, _D), jnp.float32),
            pltpu.VMEM((_A_CH, _D), jnp.float32),
            pltpu.VMEM((_A_CH,), jnp.int32),
            pltpu.VMEM((_A_CH,), jnp.int32),
        ],
    )(x1, src, dst)


_C_CH = 200
_FPT = _N_FACES // (_NC * _NS)


def _face_gather_body(x1_hbm, f0_hbm, f1_hbm, f2_hbm, u2_hbm, w_hbm,
                      f0b, f1b, f2b, A, T, B, sem, sem2, sem3, sem4):
    c = lax.axis_index("c")
    s = lax.axis_index("s")
    wid = c * _NS + s
    base = wid * _FPT
    pltpu.sync_copy(f0_hbm.at[pl.ds(base, _FPT)], f0b)
    pltpu.sync_copy(f1_hbm.at[pl.ds(base, _FPT)], f1b)
    pltpu.sync_copy(f2_hbm.at[pl.ds(base, _FPT)], f2b)

    def chunk(i, carry):
        off = i * _C_CH
        cp0 = pltpu.async_copy(x1_hbm.at[f0b.at[pl.ds(off, _C_CH)]], A, sem)
        cp1 = pltpu.async_copy(x1_hbm.at[f1b.at[pl.ds(off, _C_CH)]], T, sem2)
        pltpu.sync_copy(u2_hbm.at[pl.ds(base + off, _C_CH)], B)
        cp0.wait()
        pltpu.async_copy(x1_hbm.at[f2b.at[pl.ds(off, _C_CH)]], A, sem,
                         add=True).wait()
        cp1.wait()

        def rowop(r, carry2):
            for c8 in range(_D // 16):
                sl = pl.ds(c8 * 16, 16)
                B[r, sl] = B[r, sl] - _ALPHA * (A[r, sl] - T[r, sl])
            return carry2

        lax.fori_loop(0, _C_CH, rowop, 0)
        cw0 = pltpu.async_copy(B, w_hbm.at[pl.ds(base + off, _C_CH)], sem3)
        cw2 = pltpu.async_copy(
            B, w_hbm.at[pl.ds(2 * _N_FACES + base + off, _C_CH)], sem4)
        cw0.wait()
        cw2.wait()

        def rowneg(r, carry2):
            for c8 in range(_D // 16):
                sl = pl.ds(c8 * 16, 16)
                B[r, sl] = -B[r, sl]
            return carry2

        lax.fori_loop(0, _C_CH, rowneg, 0)
        pltpu.sync_copy(B, w_hbm.at[pl.ds(_N_FACES + base + off, _C_CH)])
        return carry

    lax.fori_loop(0, _FPT // _C_CH, chunk, 0)


@jax.jit
def _face_gather(x1, f0, f1, f2, u2):
    mesh = plsc.VectorSubcoreMesh(core_axis_name="c", subcore_axis_name="s",
                                  num_cores=_NC, num_subcores=_NS)
    return pl.kernel(
        _face_gather_body,
        out_type=jax.ShapeDtypeStruct((3 * _N_FACES, _D), jnp.float32),
        mesh=mesh,
        scratch_types=[
            pltpu.VMEM((_FPT,), jnp.int32),
            pltpu.VMEM((_FPT,), jnp.int32),
            pltpu.VMEM((_FPT,), jnp.int32),
            pltpu.VMEM((_C_CH, _D), jnp.float32),
            pltpu.VMEM((_C_CH, _D), jnp.float32),
            pltpu.VMEM((_C_CH, _D), jnp.float32),
            pltpu.SemaphoreType.DMA,
            pltpu.SemaphoreType.DMA,
            pltpu.SemaphoreType.DMA,
            pltpu.SemaphoreType.DMA,
        ],
    )(x1, f0, f1, f2, u2)


_G_CH = 200
_G_EPT = _N_EDGES // (_NC * _NS)


def _edge_gather_body(v_hbm, src_hbm, dst_hbm, g_hbm,
                      db, sb, VD, VS, semd, sems):
    c = lax.axis_index("c")
    s = lax.axis_index("s")
    wid = c * _NS + s
    base = wid * _G_EPT

    def chunk(i, carry):
        off = base + i * _G_CH
        pltpu.sync_copy(dst_hbm.at[pl.ds(off, _G_CH)], db)
        pltpu.sync_copy(src_hbm.at[pl.ds(off, _G_CH)], sb)
        cpd = pltpu.async_copy(v_hbm.at[db], VD, semd)
        cps = pltpu.async_copy(v_hbm.at[sb], VS, sems)
        cpd.wait()
        cps.wait()

        def rowop(r, carry2):
            for c8 in range(_D // 16):
                sl = pl.ds(c8 * 16, 16)
                VD[r, sl] = VD[r, sl] - VS[r, sl]
            return carry2

        lax.fori_loop(0, _G_CH, rowop, 0)
        pltpu.sync_copy(VD, g_hbm.at[pl.ds(off, _G_CH)])
        return carry

    lax.fori_loop(0, _G_EPT // _G_CH, chunk, 0)


@jax.jit
def _edge_gather(v, src, dst):
    mesh = plsc.VectorSubcoreMesh(core_axis_name="c", subcore_axis_name="s",
                                  num_cores=_NC, num_subcores=_NS)
    return pl.kernel(
        _edge_gather_body,
        out_type=jax.ShapeDtypeStruct((_N_EDGES, _D), jnp.float32),
        mesh=mesh,
        scratch_types=[
            pltpu.VMEM((_G_CH,), jnp.int32),
            pltpu.VMEM((_G_CH,), jnp.int32),
            pltpu.VMEM((_G_CH, _D), jnp.float32),
            pltpu.VMEM((_G_CH, _D), jnp.float32),
            pltpu.SemaphoreType.DMA,
            pltpu.SemaphoreType.DMA,
        ],
    )(v, src, dst)


_DC = 12800
_NCHUNK = _N_EDGES // _DC
_FPAD = 160256
_FPT_D = _FPAD // (_NC * _NS)
_SCAN_BLKS = [32] * 9 + [25]
_DSIGNS = (1.0, -1.0, 1.0)


def _edge_scatter_body(f0_hbm, f1_hbm, f2_hbm, w_hbm, e_hbm,
                       e_acc, fb0, fb1, fb2, locl, fidl, rowbuf, zbuf, cntb, sem):
    c = lax.axis_index("c")
    s = lax.axis_index("s")
    wid = c * _NS + s
    fbase = wid * _FPT_D
    pltpu.sync_copy(f0_hbm.at[pl.ds(fbase, _FPT_D)], fb0)
    pltpu.sync_copy(f1_hbm.at[pl.ds(fbase, _FPT_D)], fb1)
    pltpu.sync_copy(f2_hbm.at[pl.ds(fbase, _FPT_D)], fb2)

    def zrow(r, carry2):
        for c8 in range(_D // 16):
            zbuf[r, pl.ds(c8 * 16, 16)] = jnp.zeros((16,), jnp.float32)
        return carry2

    lax.fori_loop(0, 32, zrow, 0)
    iota16 = lax.iota(jnp.int32, 16)
    rows_pt = _DC // _NS

    def flush(k, cnt):
        plsc.store_scatter(locl, [cnt + iota16],
                           jnp.full((16,), _DC, jnp.int32) + iota16)
        plsc.store_scatter(fidl, [cnt + iota16], iota16)
        nb = lax.shift_right_logical(cnt + 15, 4)

        def batch(jb, carry3):
            fidv = fidl[pl.ds(jb * 16, 16)]
            locv = locl[pl.ds(jb * 16, 16)]
            pltpu.async_copy(w_hbm.at[fidv], rowbuf, sem).wait()
            if k == 1:
                for r in range(16):
                    for c8 in range(_D // 16):
                        sl = pl.ds(c8 * 16, 16)
                        rowbuf[r, sl] = -rowbuf[r, sl]
            pltpu.sync_copy(rowbuf, e_acc.at[locv], add=True)
            return carry3

        lax.fori_loop(0, nb, batch, 0)

    n_pass = jnp.where(c == 0, 13, _NCHUNK - 13)

    def pass_body(i, carry):
        ch = i + c * 13
        base = ch * _DC
        for j in range(rows_pt // 32):
            pltpu.sync_copy(zbuf, e_acc.at[pl.ds(s * rows_pt + j * 32, 32)])
        plsc.subcore_barrier()
        for k, fb in ((0, fb0), (1, fb1), (2, fb2)):
            voff = 0
            for nv in _SCAN_BLKS:
                def scan_iter(j, cnt, _voff=voff, _fb=fb):
                    idx = _fb[pl.ds((_voff + j) * 16, 16)]
                    loc = idx - base
                    mask = (idx >= base) & (loc < _DC)
                    cs = plsc.cumsum(mask.astype(jnp.int32))
                    pos = jnp.where(mask, cnt + cs - 1, 543)
                    plsc.store_scatter(locl, [pos], loc)
                    fidv = fbase + (_voff + j) * 16 + iota16
                    plsc.store_scatter(fidl, [pos], fidv)
                    return cnt + cs[15]

                cnt = lax.fori_loop(0, nv, scan_iter, jnp.int32(0))
                flush(k, cnt)
                voff += nv
        plsc.subcore_barrier()
        pltpu.sync_copy(e_acc.at[pl.ds(s * rows_pt, rows_pt)],
                        e_hbm.at[pl.ds(base + s * rows_pt, rows_pt)])
        return carry

    lax.fori_loop(0, n_pass, pass_body, 0)


@jax.jit
def _edge_scatter(f0p, f1p, f2p, w):
    mesh = plsc.VectorSubcoreMesh(core_axis_name="c", subcore_axis_name="s",
                                  num_cores=_NC, num_subcores=_NS)
    return pl.kernel(
        _edge_scatter_body,
        out_type=jax.ShapeDtypeStruct((_N_EDGES, _D), jnp.float32),
        mesh=mesh,
        scratch_types=[
            pltpu.VMEM_SHARED((_DC + 16, _D), jnp.float32),
            pltpu.VMEM((_FPT_D,), jnp.int32),
            pltpu.VMEM((_FPT_D,), jnp.int32),
            pltpu.VMEM((_FPT_D,), jnp.int32),
            pltpu.VMEM((544,), jnp.int32),
            pltpu.VMEM((544,), jnp.int32),
            pltpu.VMEM((16, _D), jnp.float32),
            pltpu.VMEM((32, _D), jnp.float32),
            pltpu.VMEM((16,), jnp.int32),
            pltpu.SemaphoreType.DMA,
        ],
    )(f0p, f1p, f2p, w)


def _v_body(x0_ref, yp_ref, w_ref, o_ref):
    u0 = jnp.dot(x0_ref[...], w_ref[...], preferred_element_type=jnp.float32)
    o_ref[...] = u0 - _ALPHA * (yp_ref[0] + yp_ref[1])


@jax.jit
def _v_table(x0, yp, W_grad_t):
    n = x0.shape[0]
    return pl.pallas_call(
        _v_body,
        grid=(n // _BLK,),
        in_specs=[
            pl.BlockSpec((_BLK, _D), lambda i: (i, 0)),
            pl.BlockSpec((_NC, _BLK, _D), lambda i: (0, i, 0)),
            pl.BlockSpec((_D, _D), lambda i: (0, 0)),
        ],
        out_specs=pl.BlockSpec((_BLK, _D), lambda i: (i, 0)),
        out_shape=jax.ShapeDtypeStruct((n, _D), jnp.float32),
    )(x0, yp, W_grad_t)


def _mm_body(x_ref, w_ref, o_ref):
    o_ref[...] = jnp.dot(x_ref[...], w_ref[...], preferred_element_type=jnp.float32)


@jax.jit
def _matmul(x, Wt):
    n = x.shape[0]
    return pl.pallas_call(
        _mm_body,
        grid=(n // _BLK,),
        in_specs=[
            pl.BlockSpec((_BLK, _D), lambda i: (i, 0)),
            pl.BlockSpec((_D, _D), lambda i: (0, 0)),
        ],
        out_specs=pl.BlockSpec((_BLK, _D), lambda i: (i, 0)),
        out_shape=jax.ShapeDtypeStruct((n, _D), jnp.float32),
    )(x, Wt)


def _final_body(x1_ref, e_ref, w_ref, b_ref, o_ref):
    acc = jnp.dot(x1_ref[...], w_ref[...], preferred_element_type=jnp.float32)
    o_ref[...] = jnp.maximum(acc + e_ref[...] + b_ref[...], 0.0)


@jax.jit
def _final_matmul(x1, e, W_self_t, bias):
    n = x1.shape[0]
    return pl.pallas_call(
        _final_body,
        grid=(n // _BLK,),
        in_specs=[
            pl.BlockSpec((_BLK, _D), lambda i: (i, 0)),
            pl.BlockSpec((_BLK, _D), lambda i: (i, 0)),
            pl.BlockSpec((_D, _D), lambda i: (0, 0)),
            pl.BlockSpec((1, _D), lambda i: (0, 0)),
        ],
        out_specs=pl.BlockSpec((_BLK, _D), lambda i: (i, 0)),
        out_shape=jax.ShapeDtypeStruct((n, _D), jnp.float32),
    )(x1, e, W_self_t, bias)


def kernel(x1, x0, x2, edge_index, face_edge_index, W_self, W_grad, W_curl, bias):
    src = edge_index[0]
    dst = edge_index[1]
    fe0, fe1, fe2 = face_edge_index[0], face_edge_index[1], face_edge_index[2]

    yp = _node_scatter(x1, src, dst)
    v = _v_table(x0, yp, W_grad.T)
    g = _edge_gather(v, src, dst)

    u2 = _matmul(x2, W_curl.T)
    wsigned = _face_gather(x1, fe0, fe1, fe2, u2)

    all_idx = jnp.concatenate([fe0, fe1, fe2])
    e = g.at[all_idx].add(wsigned)

    return _final_matmul(x1, e, W_self.T, bias.reshape(1, _D))
